# baseline (speedup 1.0000x reference)
import numpy as np
import jax
import jax.numpy as jnp
from jax import lax
from jax.experimental import pallas as pl
from jax.experimental.pallas import tpu as pltpu
from jax.experimental.pallas import tpu_sc as plsc

_F = 26
_VOCAB = 26000
_D = 64
_B = 4096
_NC, _NS, _L = 2, 16, 16
_NW = _NC * _NS
_BPW = _B // _NW
_NPAIR = _F * (_F - 1) // 2
_BOFF = 328
_NSLOT = 2 * _BOFF
_CHUNKS = ((0, 128), (128, 128), (256, 72), (328, 128), (456, 128), (584, 72))


def _build_consts():
    base = np.zeros(_NSLOT, np.int32)
    kmod = np.zeros(_NSLOT, np.int32)
    p = 0
    for i in range(_F - 1):
        for j in range(i + 1, _F):
            base[p] = j * _VOCAB + i * 1000
            kmod[p] = i
            base[_BOFF + p] = i * _VOCAB + j * 1000
            kmod[_BOFF + p] = j
            p += 1
    return base, kmod


_BASE_NP, _KMOD_NP = _build_consts()


def _ffm_body(x_hbm, w_hbm, base_hbm, kmod_hbm, out_hbm,
              xs_v, base_v, kmod_v, idx0_v, idx1_v, rows0_v, rows1_v,
              part_v, out_v, sem0, sem1):
    wid = lax.axis_index("s") * _NC + lax.axis_index("c")
    sbase = wid * _BPW

    pltpu.sync_copy(x_hbm.at[pl.ds(sbase * _F, _BPW * _F)], xs_v)
    pltpu.sync_copy(base_hbm, base_v)
    pltpu.sync_copy(kmod_hbm, kmod_v)

    def build_idx(s, idx_v):
        for c in range(_NSLOT // _L):
            sl = pl.ds(c * _L, _L)
            km = kmod_v[sl]
            xv = plsc.load_gather(xs_v, [s * _F + km])
            idx_v[sl] = base_v[sl] + xv

    def fire(idx_v, rows_v, sem):
        for s0, n in _CHUNKS:
            pltpu.async_copy(
                w_hbm.at[idx_v.at[pl.ds(s0, n)]],
                rows_v.at[pl.ds(s0, n), :],
                sem,
            )

    def drain(idx_v, rows_v, sem):
        for s0, n in _CHUNKS:
            pltpu.make_async_copy(
                w_hbm.at[idx_v.at[pl.ds(s0, n)]],
                rows_v.at[pl.ds(s0, n), :],
                sem,
            ).wait()

    def compute(rows_v, s):
        def pair_body(p, accs):
            return tuple(
                accs[q]
                + rows_v[p, pl.ds(q * _L, _L)]
                * rows_v[_BOFF + p, pl.ds(q * _L, _L)]
                for q in range(_D // _L)
            )

        zero = jnp.zeros((_L,), jnp.float32)
        accs = lax.fori_loop(0, _NPAIR, pair_body, (zero,) * 4, unroll=8)
        part_v[pl.ds(s * _L, _L)] = accs[0] + accs[1] + accs[2] + accs[3]

    build_idx(0, idx0_v)
    fire(idx0_v, rows0_v, sem0)
    build_idx(1, idx1_v)
    fire(idx1_v, rows1_v, sem1)

    @pl.loop(0, _BPW // 2)
    def _(t):
        s = t * 2
        drain(idx0_v, rows0_v, sem0)
        compute(rows0_v, s)
        s2 = jnp.minimum(s + 2, _BPW - 1)
        build_idx(s2, idx0_v)
        fire(idx0_v, rows0_v, sem0)
        drain(idx1_v, rows1_v, sem1)
        compute(rows1_v, s + 1)
        s3 = jnp.minimum(s + 3, _BPW - 1)
        build_idx(s3, idx1_v)
        fire(idx1_v, rows1_v, sem1)

    drain(idx0_v, rows0_v, sem0)
    drain(idx1_v, rows1_v, sem1)

    lanes = lax.iota(jnp.int32, _L)
    for g in range(_BPW // _L):
        acc = jnp.zeros((_L,), jnp.float32)
        for c in range(_L):
            gidx = g * _L * _L + lanes * _L + c
            acc = acc + plsc.load_gather(part_v, [gidx])
        out_v[pl.ds(g * _L, _L)] = acc

    pltpu.sync_copy(out_v, out_hbm.at[pl.ds(sbase, _BPW)])


@jax.jit
def kernel(x, W):
    xi = x.astype(jnp.int32).reshape(_B * _F)
    wf = W.reshape(_F * _VOCAB, _D)
    mesh = plsc.VectorSubcoreMesh(
        core_axis_name="c", subcore_axis_name="s",
        num_cores=_NC, num_subcores=_NS,
    )
    run = pl.kernel(
        _ffm_body,
        out_type=jax.ShapeDtypeStruct((_B,), jnp.float32),
        mesh=mesh,
        compiler_params=pltpu.CompilerParams(
            needs_layout_passes=False, use_tc_tiling_on_sc=False,
        ),
        scratch_types=[
            pltpu.VMEM((_BPW * _F,), jnp.int32),
            pltpu.VMEM((_NSLOT,), jnp.int32),
            pltpu.VMEM((_NSLOT,), jnp.int32),
            pltpu.VMEM((_NSLOT,), jnp.int32),
            pltpu.VMEM((_NSLOT,), jnp.int32),
            pltpu.VMEM((_NSLOT, _D), jnp.float32),
            pltpu.VMEM((_NSLOT, _D), jnp.float32),
            pltpu.VMEM((_BPW * _L,), jnp.float32),
            pltpu.VMEM((_BPW,), jnp.float32),
            pltpu.SemaphoreType.DMA,
            pltpu.SemaphoreType.DMA,
        ],
    )
    out = run(xi, wf, jnp.asarray(_BASE_NP), jnp.asarray(_KMOD_NP))
    return out[:, None]


# speedup vs baseline: 1.0047x; 1.0047x over previous
import numpy as np
import jax
import jax.numpy as jnp
from jax import lax
from jax.experimental import pallas as pl
from jax.experimental.pallas import tpu as pltpu
from jax.experimental.pallas import tpu_sc as plsc

_F = 26
_VOCAB = 26000
_D = 64
_B = 4096
_NC, _NS, _L = 2, 16, 16
_NW = _NC * _NS
_BPW = _B // _NW
_NPAIR = _F * (_F - 1) // 2
_BOFF = 328
_NSLOT = 2 * _BOFF
_CHUNKS = ((0, 128), (128, 128), (256, 72), (328, 128), (456, 128), (584, 72))


def _build_consts():
    base = np.zeros(_NSLOT, np.int32)
    kmod = np.zeros(_NSLOT, np.int32)
    p = 0
    for i in range(_F - 1):
        for j in range(i + 1, _F):
            base[p] = j * _VOCAB + i * 1000
            kmod[p] = i
            base[_BOFF + p] = i * _VOCAB + j * 1000
            kmod[_BOFF + p] = j
            p += 1
    return base, kmod


_BASE_NP, _KMOD_NP = _build_consts()


def _ffm_body(x_hbm, w_hbm, base_hbm, kmod_hbm, out_hbm,
              xs_v, base_v, kmod_v, idx0_v, idx1_v, rows0_v, rows1_v,
              part_v, out_v, sem0, sem1):
    wid = lax.axis_index("s") * _NC + lax.axis_index("c")
    sbase = wid * _BPW

    pltpu.sync_copy(x_hbm.at[pl.ds(sbase * _F, _BPW * _F)], xs_v)
    pltpu.sync_copy(base_hbm, base_v)
    pltpu.sync_copy(kmod_hbm, kmod_v)

    def build_idx(s, idx_v):
        for c in range(_NSLOT // _L):
            sl = pl.ds(c * _L, _L)
            km = kmod_v[sl]
            xv = plsc.load_gather(xs_v, [s * _F + km])
            idx_v[sl] = base_v[sl] + xv

    def fire(idx_v, rows_v, sem):
        for s0, n in _CHUNKS:
            pltpu.async_copy(
                w_hbm.at[idx_v.at[pl.ds(s0, n)]],
                rows_v.at[pl.ds(s0, n), :],
                sem,
            )

    def drain(idx_v, rows_v, sem):
        for s0, n in _CHUNKS:
            pltpu.make_async_copy(
                w_hbm.at[idx_v.at[pl.ds(s0, n)]],
                rows_v.at[pl.ds(s0, n), :],
                sem,
            ).wait()

    def compute(rows_v, s):
        def pair_body(p, accs):
            return tuple(
                accs[q]
                + rows_v[p, pl.ds(q * _L, _L)]
                * rows_v[_BOFF + p, pl.ds(q * _L, _L)]
                for q in range(_D // _L)
            )

        zero = jnp.zeros((_L,), jnp.float32)
        accs = lax.fori_loop(0, _NPAIR, pair_body, (zero,) * 4, unroll=8)
        part_v[pl.ds(s * _L, _L)] = accs[0] + accs[1] + accs[2] + accs[3]

    build_idx(0, idx0_v)
    fire(idx0_v, rows0_v, sem0)

    @pl.loop(0, _BPW // 2)
    def _(t):
        s = t * 2
        build_idx(s + 1, idx1_v)
        fire(idx1_v, rows1_v, sem1)
        drain(idx0_v, rows0_v, sem0)
        compute(rows0_v, s)
        s2 = jnp.minimum(s + 2, _BPW - 1)
        build_idx(s2, idx0_v)
        fire(idx0_v, rows0_v, sem0)
        drain(idx1_v, rows1_v, sem1)
        compute(rows1_v, s + 1)

    drain(idx0_v, rows0_v, sem0)

    lanes = lax.iota(jnp.int32, _L)
    for g in range(_BPW // _L):
        acc = jnp.zeros((_L,), jnp.float32)
        for c in range(_L):
            gidx = g * _L * _L + lanes * _L + c
            acc = acc + plsc.load_gather(part_v, [gidx])
        out_v[pl.ds(g * _L, _L)] = acc

    pltpu.sync_copy(out_v, out_hbm.at[pl.ds(sbase, _BPW)])


@jax.jit
def kernel(x, W):
    xi = x.astype(jnp.int32).reshape(_B * _F)
    wf = W.reshape(_F * _VOCAB, _D)
    mesh = plsc.VectorSubcoreMesh(
        core_axis_name="c", subcore_axis_name="s",
        num_cores=_NC, num_subcores=_NS,
    )
    run = pl.kernel(
        _ffm_body,
        out_type=jax.ShapeDtypeStruct((_B,), jnp.float32),
        mesh=mesh,
        compiler_params=pltpu.CompilerParams(
            needs_layout_passes=False, use_tc_tiling_on_sc=False,
        ),
        scratch_types=[
            pltpu.VMEM((_BPW * _F,), jnp.int32),
            pltpu.VMEM((_NSLOT,), jnp.int32),
            pltpu.VMEM((_NSLOT,), jnp.int32),
            pltpu.VMEM((_NSLOT,), jnp.int32),
            pltpu.VMEM((_NSLOT,), jnp.int32),
            pltpu.VMEM((_NSLOT, _D), jnp.float32),
            pltpu.VMEM((_NSLOT, _D), jnp.float32),
            pltpu.VMEM((_BPW * _L,), jnp.float32),
            pltpu.VMEM((_BPW,), jnp.float32),
            pltpu.SemaphoreType.DMA,
            pltpu.SemaphoreType.DMA,
        ],
    )
    out = run(xi, wf, jnp.asarray(_BASE_NP), jnp.asarray(_KMOD_NP))
    return out[:, None]
